# baseline (speedup 1.0000x reference)
import functools

import jax
import jax.numpy as jnp
from jax import lax
from jax.experimental import pallas as pl
from jax.experimental.pallas import tpu as pltpu
from jax.experimental.pallas import tpu_sc as plsc

N_NODES = 10000
D = 128
DH = D // 2
N_PAD = 10240
E_PAD = 327680
EW = E_PAD // 16
ERW = EW // 128
CHUNK = 256
CR = CHUNK // 128
NCHUNK = EW // CHUNK
NRING = 6
NBUFS = 5
ZROWS = 128
TROWS = N_PAD // 16


def _build_sc_agg():
    mesh = plsc.VectorSubcoreMesh(core_axis_name="c", subcore_axis_name="s")

    @functools.partial(
        pl.kernel,
        mesh=mesh,
        compiler_params=pltpu.CompilerParams(use_tc_tiling_on_sc=False),
        out_type=jax.ShapeDtypeStruct((2, N_PAD, DH), jnp.float32),
        scratch_types=[
            pltpu.VMEM((NRING, CHUNK), jnp.int32),
            pltpu.VMEM((NRING, CHUNK), jnp.int32),
            pltpu.VMEM((NRING, CHUNK), jnp.float32),
            pltpu.VMEM((CHUNK, DH), jnp.float32),
            pltpu.VMEM((CHUNK, DH), jnp.float32),
            pltpu.VMEM((CHUNK, DH), jnp.float32),
            pltpu.VMEM((CHUNK, DH), jnp.float32),
            pltpu.VMEM((CHUNK, DH), jnp.float32),
            pltpu.VMEM_SHARED((N_PAD, DH), jnp.float32),
            pltpu.SemaphoreType.DMA,
            pltpu.SemaphoreType.DMA,
            pltpu.SemaphoreType.DMA,
        ],
    )
    def sc_agg(xh_hbm, col_hbm, row_hbm, a_hbm, out_hbm,
               col_v, row_v, a_v, rows0_v, rows1_v, rows2_v, rows3_v, rows4_v,
               agg_sh, sem_i, sem_g, sem_s):
        cid = lax.axis_index("c")
        sid = lax.axis_index("s")
        zero16 = jnp.zeros((16,), jnp.float32)
        cbias16 = jnp.full((16,), 1, jnp.int32) * cid
        rows_bufs = (rows0_v, rows1_v, rows2_v, rows3_v, rows4_v)

        def issue_idx(c):
            slot = lax.rem(c, NRING)
            ebase = sid * EW + c * CHUNK
            pltpu.async_copy(col_hbm.at[pl.ds(ebase, CHUNK)],
                             col_v.at[slot], sem_i)
            pltpu.async_copy(row_hbm.at[pl.ds(ebase, CHUNK)],
                             row_v.at[slot], sem_i)
            pltpu.async_copy(a_hbm.at[pl.ds(ebase, CHUNK)],
                             a_v.at[slot], sem_i)

        def wait_idx(c):
            slot = lax.rem(c, NRING)
            ebase = sid * EW + c * CHUNK
            pltpu.make_async_copy(col_hbm.at[pl.ds(ebase, CHUNK)],
                                  col_v.at[slot], sem_i).wait()
            pltpu.make_async_copy(row_hbm.at[pl.ds(ebase, CHUNK)],
                                  row_v.at[slot], sem_i).wait()
            pltpu.make_async_copy(a_hbm.at[pl.ds(ebase, CHUNK)],
                                  a_v.at[slot], sem_i).wait()

        def bias_cols(c):
            slot = lax.rem(c, NRING)
            for j in range(CHUNK // 16):
                cv = col_v[slot, pl.ds(j * 16, 16)]
                col_v[slot, pl.ds(j * 16, 16)] = cv + cv + cbias16

        def issue_gathers(c, buf):
            slot = lax.rem(c, NRING)
            pltpu.async_copy(xh_hbm.at[col_v.at[slot]], buf, sem_g)

        def wait_gathers(c, buf):
            slot = lax.rem(c, NRING)
            pltpu.make_async_copy(xh_hbm.at[col_v.at[slot]], buf,
                                  sem_g).wait()

        def issue_scatters(c, buf):
            slot = lax.rem(c, NRING)
            pltpu.async_copy(buf, agg_sh.at[row_v.at[slot]], sem_s, add=True)

        def wait_scatters(c, buf):
            slot = lax.rem(c, NRING)
            pltpu.make_async_copy(buf, agg_sh.at[row_v.at[slot]],
                                  sem_s).wait()

        issue_idx(0)
        issue_idx(1)
        issue_idx(2)

        def zrow(i, carry):
            for j in range(DH // 16):
                rows0_v[i, pl.ds(j * 16, 16)] = zero16
            return carry

        lax.fori_loop(0, ZROWS, zrow, 0)
        tile_row0 = sid * TROWS
        for k in range(TROWS // ZROWS):
            pltpu.sync_copy(rows0_v.at[pl.ds(0, ZROWS)],
                            agg_sh.at[pl.ds(tile_row0 + k * ZROWS, ZROWS)])
        wait_idx(0)
        bias_cols(0)
        issue_gathers(0, rows0_v)
        wait_idx(1)
        bias_cols(1)
        issue_gathers(1, rows1_v)
        plsc.subcore_barrier()

        _dnums = lax.GatherDimensionNumbers(
            offset_dims=(), collapsed_slice_dims=(0,), start_index_map=(0,))

        def scale(c, buf):
            slot = lax.rem(c, NRING)
            nj = DH // 16

            def grp(g, carry2):
                a16 = a_v[slot, pl.ds(g * 16, 16)]
                for l0 in range(0, 16, 4):
                    es = [g * 16 + l0 + i for i in range(4)]
                    bcs = [
                        lax.gather(
                            a16, jnp.full((16, 1), l0 + i, jnp.int32),
                            _dnums, (1,),
                            mode=lax.GatherScatterMode.PROMISE_IN_BOUNDS)
                        for i in range(4)
                    ]
                    vals = [[buf[es[i], pl.ds(j * 16, 16)] for j in range(nj)]
                            for i in range(4)]
                    for i in range(4):
                        for j in range(nj):
                            buf[es[i], pl.ds(j * 16, 16)] = vals[i][j] * bcs[i]
                return carry2

            lax.fori_loop(0, CHUNK // 16, grp, 0)

        def body5(c5, carry):
            for b in range(NBUFS):
                c = c5 * NBUFS + b
                buf = rows_bufs[b]
                gbuf = rows_bufs[(b + 2) % NBUFS]
                wait_gathers(c, buf)

                @pl.when(c + 2 < NCHUNK)
                def _():
                    @pl.when(c >= 3)
                    def _():
                        wait_scatters(c - 3, gbuf)

                    wait_idx(c + 2)
                    bias_cols(c + 2)
                    issue_gathers(c + 2, gbuf)

                @pl.when(c + 3 < NCHUNK)
                def _():
                    issue_idx(c + 3)

                scale(c, buf)
                issue_scatters(c, buf)
            return carry

        lax.fori_loop(0, NCHUNK // NBUFS, body5, 0)
        for c in range(NCHUNK - 5, NCHUNK):
            wait_scatters(c, rows_bufs[c % NBUFS])

        plsc.subcore_barrier()
        for k in range(TROWS // ZROWS):
            r0 = tile_row0 + k * ZROWS
            pltpu.sync_copy(agg_sh.at[pl.ds(r0, ZROWS)],
                            rows0_v.at[pl.ds(0, ZROWS)])
            pltpu.sync_copy(rows0_v.at[pl.ds(0, ZROWS)],
                            out_hbm.at[cid, pl.ds(r0, ZROWS)])

    return sc_agg


_sc_agg = _build_sc_agg()

_MB = 1000


def _tc_body(agg_ref, w_ref, b_ref, g_ref, bt_ref, o_ref):
    acc = jnp.concatenate([agg_ref[0], agg_ref[1]], axis=-1)
    h = jnp.dot(acc, w_ref[...], preferred_element_type=jnp.float32)
    h = jnp.maximum(h + b_ref[...], 0.0)
    o_ref[...] = (g_ref[...] * h) / jnp.sqrt(jnp.float32(1.0 + 1e-3)) + bt_ref[...]


def _tc_finish(partials, W, b, gamma, beta):
    return pl.pallas_call(
        _tc_body,
        grid=(N_NODES // _MB,),
        in_specs=[
            pl.BlockSpec((2, _MB, DH), lambda i: (0, i, 0)),
            pl.BlockSpec((D, D), lambda i: (0, 0)),
            pl.BlockSpec((1, D), lambda i: (0, 0)),
            pl.BlockSpec((1, D), lambda i: (0, 0)),
            pl.BlockSpec((1, D), lambda i: (0, 0)),
        ],
        out_specs=pl.BlockSpec((_MB, D), lambda i: (i, 0)),
        out_shape=jax.ShapeDtypeStruct((N_NODES, D), jnp.float32),
    )(partials, W, b.reshape(1, D), gamma.reshape(1, D), beta.reshape(1, D))


def kernel(x, edge_index, a_values, W, b, gamma, beta):
    row = edge_index[0].astype(jnp.int32)
    col = edge_index[1].astype(jnp.int32)
    n_edges = row.shape[0]
    pad = E_PAD - n_edges
    ipad = jnp.zeros((pad,), jnp.int32)
    rowp = jnp.concatenate([row, ipad])
    colp = jnp.concatenate([col, ipad])
    ap = jnp.concatenate([a_values, jnp.zeros((pad,), jnp.float32)])
    xh = x.reshape(2 * N_NODES, DH)
    partials = _sc_agg(xh, colp, rowp, ap)
    return _tc_finish(partials, W, b, gamma, beta)


# speedup vs baseline: 1.1590x; 1.1590x over previous
import functools

import jax
import jax.numpy as jnp
from jax import lax
from jax.experimental import pallas as pl
from jax.experimental.pallas import tpu as pltpu
from jax.experimental.pallas import tpu_sc as plsc

N_NODES = 10000
D = 128
DH = D // 2
N_PAD = 10240
E_PAD = 327680
EW = E_PAD // 16
ERW = EW // 128
CHUNK = 256
CR = CHUNK // 128
NCHUNK = EW // CHUNK
NRING = 6
NBUFS = 5
ZROWS = 128
TROWS = N_PAD // 16


def _build_sc_agg():
    mesh = plsc.VectorSubcoreMesh(core_axis_name="c", subcore_axis_name="s")

    @functools.partial(
        pl.kernel,
        mesh=mesh,
        compiler_params=pltpu.CompilerParams(use_tc_tiling_on_sc=False),
        out_type=jax.ShapeDtypeStruct((2, N_PAD, DH), jnp.float32),
        scratch_types=[
            pltpu.VMEM((NRING, CHUNK), jnp.int32),
            pltpu.VMEM((NRING, CHUNK), jnp.int32),
            pltpu.VMEM((NRING, CHUNK), jnp.float32),
            pltpu.VMEM((CHUNK, DH), jnp.float32),
            pltpu.VMEM((CHUNK, DH), jnp.float32),
            pltpu.VMEM((CHUNK, DH), jnp.float32),
            pltpu.VMEM((CHUNK, DH), jnp.float32),
            pltpu.VMEM((CHUNK, DH), jnp.float32),
            pltpu.VMEM_SHARED((N_PAD, DH), jnp.float32),
            pltpu.SemaphoreType.DMA,
            pltpu.SemaphoreType.DMA,
            pltpu.SemaphoreType.DMA,
        ],
    )
    def sc_agg(xh_hbm, col_hbm, row_hbm, a_hbm, out_hbm,
               col_v, row_v, a_v, rows0_v, rows1_v, rows2_v, rows3_v, rows4_v,
               agg_sh, sem_i, sem_g, sem_s):
        cid = lax.axis_index("c")
        sid = lax.axis_index("s")
        zero16 = jnp.zeros((16,), jnp.float32)
        cbias16 = jnp.full((16,), N_NODES, jnp.int32) * cid
        rows_bufs = (rows0_v, rows1_v, rows2_v, rows3_v, rows4_v)

        def issue_idx(c):
            slot = lax.rem(c, NRING)
            ebase = sid * EW + c * CHUNK
            pltpu.async_copy(col_hbm.at[pl.ds(ebase, CHUNK)],
                             col_v.at[slot], sem_i)
            pltpu.async_copy(row_hbm.at[pl.ds(ebase, CHUNK)],
                             row_v.at[slot], sem_i)
            pltpu.async_copy(a_hbm.at[pl.ds(ebase, CHUNK)],
                             a_v.at[slot], sem_i)

        def wait_idx(c):
            slot = lax.rem(c, NRING)
            ebase = sid * EW + c * CHUNK
            pltpu.make_async_copy(col_hbm.at[pl.ds(ebase, CHUNK)],
                                  col_v.at[slot], sem_i).wait()
            pltpu.make_async_copy(row_hbm.at[pl.ds(ebase, CHUNK)],
                                  row_v.at[slot], sem_i).wait()
            pltpu.make_async_copy(a_hbm.at[pl.ds(ebase, CHUNK)],
                                  a_v.at[slot], sem_i).wait()

        def bias_cols(c):
            slot = lax.rem(c, NRING)
            for j in range(CHUNK // 16):
                col_v[slot, pl.ds(j * 16, 16)] = (
                    col_v[slot, pl.ds(j * 16, 16)] + cbias16)

        def issue_gathers(c, buf):
            slot = lax.rem(c, NRING)
            pltpu.async_copy(xh_hbm.at[col_v.at[slot]], buf, sem_g)

        def wait_gathers(c, buf):
            slot = lax.rem(c, NRING)
            pltpu.make_async_copy(xh_hbm.at[col_v.at[slot]], buf,
                                  sem_g).wait()

        def issue_scatters(c, buf):
            slot = lax.rem(c, NRING)
            pltpu.async_copy(buf, agg_sh.at[row_v.at[slot]], sem_s, add=True)

        def wait_scatters(c, buf):
            slot = lax.rem(c, NRING)
            pltpu.make_async_copy(buf, agg_sh.at[row_v.at[slot]],
                                  sem_s).wait()

        issue_idx(0)
        issue_idx(1)
        issue_idx(2)

        def zrow(i, carry):
            for j in range(DH // 16):
                rows0_v[i, pl.ds(j * 16, 16)] = zero16
            return carry

        lax.fori_loop(0, ZROWS, zrow, 0)
        tile_row0 = sid * TROWS
        for k in range(TROWS // ZROWS):
            pltpu.sync_copy(rows0_v.at[pl.ds(0, ZROWS)],
                            agg_sh.at[pl.ds(tile_row0 + k * ZROWS, ZROWS)])
        wait_idx(0)
        bias_cols(0)
        issue_gathers(0, rows0_v)
        wait_idx(1)
        bias_cols(1)
        issue_gathers(1, rows1_v)
        plsc.subcore_barrier()

        _dnums = lax.GatherDimensionNumbers(
            offset_dims=(), collapsed_slice_dims=(0,), start_index_map=(0,))

        def scale(c, buf):
            slot = lax.rem(c, NRING)
            nj = DH // 16

            def grp(g, carry2):
                a16 = a_v[slot, pl.ds(g * 16, 16)]
                for l0 in range(0, 16, 4):
                    es = [g * 16 + l0 + i for i in range(4)]
                    bcs = [
                        lax.gather(
                            a16, jnp.full((16, 1), l0 + i, jnp.int32),
                            _dnums, (1,),
                            mode=lax.GatherScatterMode.PROMISE_IN_BOUNDS)
                        for i in range(4)
                    ]
                    vals = [[buf[es[i], pl.ds(j * 16, 16)] for j in range(nj)]
                            for i in range(4)]
                    for i in range(4):
                        for j in range(nj):
                            buf[es[i], pl.ds(j * 16, 16)] = vals[i][j] * bcs[i]
                return carry2

            lax.fori_loop(0, CHUNK // 16, grp, 0)

        def body5(c5, carry):
            for b in range(NBUFS):
                c = c5 * NBUFS + b
                buf = rows_bufs[b]
                gbuf = rows_bufs[(b + 2) % NBUFS]
                wait_gathers(c, buf)

                @pl.when(c + 2 < NCHUNK)
                def _():
                    @pl.when(c >= 3)
                    def _():
                        wait_scatters(c - 3, gbuf)

                    wait_idx(c + 2)
                    bias_cols(c + 2)
                    issue_gathers(c + 2, gbuf)

                @pl.when(c + 3 < NCHUNK)
                def _():
                    issue_idx(c + 3)

                scale(c, buf)
                issue_scatters(c, buf)
            return carry

        lax.fori_loop(0, NCHUNK // NBUFS, body5, 0)
        for c in range(NCHUNK - 5, NCHUNK):
            wait_scatters(c, rows_bufs[c % NBUFS])

        plsc.subcore_barrier()
        for k in range(TROWS // ZROWS):
            r0 = tile_row0 + k * ZROWS
            pltpu.sync_copy(agg_sh.at[pl.ds(r0, ZROWS)],
                            rows0_v.at[pl.ds(0, ZROWS)])
            pltpu.sync_copy(rows0_v.at[pl.ds(0, ZROWS)],
                            out_hbm.at[cid, pl.ds(r0, ZROWS)])

    return sc_agg


_sc_agg = _build_sc_agg()

_MB = 1000


def _tc_body(agg_ref, w_ref, b_ref, g_ref, bt_ref, o_ref):
    acc = jnp.concatenate([agg_ref[0], agg_ref[1]], axis=-1)
    h = jnp.dot(acc, w_ref[...], preferred_element_type=jnp.float32)
    h = jnp.maximum(h + b_ref[...], 0.0)
    o_ref[...] = (g_ref[...] * h) / jnp.sqrt(jnp.float32(1.0 + 1e-3)) + bt_ref[...]


def _tc_finish(partials, W, b, gamma, beta):
    return pl.pallas_call(
        _tc_body,
        grid=(N_NODES // _MB,),
        in_specs=[
            pl.BlockSpec((2, _MB, DH), lambda i: (0, i, 0)),
            pl.BlockSpec((D, D), lambda i: (0, 0)),
            pl.BlockSpec((1, D), lambda i: (0, 0)),
            pl.BlockSpec((1, D), lambda i: (0, 0)),
            pl.BlockSpec((1, D), lambda i: (0, 0)),
        ],
        out_specs=pl.BlockSpec((_MB, D), lambda i: (i, 0)),
        out_shape=jax.ShapeDtypeStruct((N_NODES, D), jnp.float32),
    )(partials, W, b.reshape(1, D), gamma.reshape(1, D), beta.reshape(1, D))


def kernel(x, edge_index, a_values, W, b, gamma, beta):
    row = edge_index[0].astype(jnp.int32)
    col = edge_index[1].astype(jnp.int32)
    n_edges = row.shape[0]
    pad = E_PAD - n_edges
    ipad = jnp.zeros((pad,), jnp.int32)
    rowp = jnp.concatenate([row, ipad])
    colp = jnp.concatenate([col, ipad])
    ap = jnp.concatenate([a_values, jnp.zeros((pad,), jnp.float32)])
    xh = jnp.concatenate([x[:, :DH], x[:, DH:]], axis=0)
    partials = _sc_agg(xh, colp, rowp, ap)
    return _tc_finish(partials, W, b, gamma, beta)


# speedup vs baseline: 1.1777x; 1.0162x over previous
import functools

import jax
import jax.numpy as jnp
from jax import lax
from jax.experimental import pallas as pl
from jax.experimental.pallas import tpu as pltpu
from jax.experimental.pallas import tpu_sc as plsc

N_NODES = 10000
D = 128
DH = D // 2
N_PAD = 10240
E_PAD = 327680
EW = E_PAD // 16
ERW = EW // 128
CHUNK = 256
CR = CHUNK // 128
NCHUNK = EW // CHUNK
NRING = 6
NBUFS = 5
ZROWS = 128
TROWS = N_PAD // 16


def _build_sc_agg():
    mesh = plsc.VectorSubcoreMesh(core_axis_name="c", subcore_axis_name="s")

    @functools.partial(
        pl.kernel,
        mesh=mesh,
        compiler_params=pltpu.CompilerParams(use_tc_tiling_on_sc=False),
        out_type=jax.ShapeDtypeStruct((2, N_PAD, DH), jnp.float32),
        scratch_types=[
            pltpu.VMEM((NRING, CHUNK), jnp.int32),
            pltpu.VMEM((NRING, CHUNK), jnp.int32),
            pltpu.VMEM((NRING, CHUNK), jnp.float32),
            pltpu.VMEM((CHUNK, DH), jnp.float32),
            pltpu.VMEM((CHUNK, DH), jnp.float32),
            pltpu.VMEM((CHUNK, DH), jnp.float32),
            pltpu.VMEM((CHUNK, DH), jnp.float32),
            pltpu.VMEM((CHUNK, DH), jnp.float32),
            pltpu.VMEM_SHARED((N_PAD, DH), jnp.float32),
            pltpu.SemaphoreType.DMA,
            pltpu.SemaphoreType.DMA,
            pltpu.SemaphoreType.DMA,
        ],
    )
    def sc_agg(xh_hbm, col_hbm, row_hbm, a_hbm, out_hbm,
               col_v, row_v, a_v, rows0_v, rows1_v, rows2_v, rows3_v, rows4_v,
               agg_sh, sem_i, sem_g, sem_s):
        cid = lax.axis_index("c")
        sid = lax.axis_index("s")
        zero16 = jnp.zeros((16,), jnp.float32)
        cbias16 = jnp.full((16,), N_NODES, jnp.int32) * cid
        rows_bufs = (rows0_v, rows1_v, rows2_v, rows3_v, rows4_v)

        def issue_idx(c):
            slot = lax.rem(c, NRING)
            ebase = sid * EW + c * CHUNK
            pltpu.async_copy(col_hbm.at[pl.ds(ebase, CHUNK)],
                             col_v.at[slot], sem_i)
            pltpu.async_copy(row_hbm.at[pl.ds(ebase, CHUNK)],
                             row_v.at[slot], sem_i)
            pltpu.async_copy(a_hbm.at[pl.ds(ebase, CHUNK)],
                             a_v.at[slot], sem_i)

        def wait_idx(c):
            slot = lax.rem(c, NRING)
            ebase = sid * EW + c * CHUNK
            pltpu.make_async_copy(col_hbm.at[pl.ds(ebase, CHUNK)],
                                  col_v.at[slot], sem_i).wait()
            pltpu.make_async_copy(row_hbm.at[pl.ds(ebase, CHUNK)],
                                  row_v.at[slot], sem_i).wait()
            pltpu.make_async_copy(a_hbm.at[pl.ds(ebase, CHUNK)],
                                  a_v.at[slot], sem_i).wait()

        def bias_cols(c):
            slot = lax.rem(c, NRING)
            for j in range(CHUNK // 16):
                col_v[slot, pl.ds(j * 16, 16)] = (
                    col_v[slot, pl.ds(j * 16, 16)] + cbias16)

        def issue_gathers(c, buf):
            slot = lax.rem(c, NRING)
            pltpu.async_copy(xh_hbm.at[col_v.at[slot]], buf, sem_g)

        def wait_gathers(c, buf):
            slot = lax.rem(c, NRING)
            pltpu.make_async_copy(xh_hbm.at[col_v.at[slot]], buf,
                                  sem_g).wait()

        def issue_scatters(c, buf):
            slot = lax.rem(c, NRING)
            pltpu.async_copy(buf, agg_sh.at[row_v.at[slot]], sem_s, add=True)

        def wait_scatters(c, buf):
            slot = lax.rem(c, NRING)
            pltpu.make_async_copy(buf, agg_sh.at[row_v.at[slot]],
                                  sem_s).wait()

        issue_idx(0)
        issue_idx(1)
        issue_idx(2)

        def zrow(i, carry):
            for j in range(DH // 16):
                rows0_v[i, pl.ds(j * 16, 16)] = zero16
            return carry

        lax.fori_loop(0, ZROWS, zrow, 0)
        tile_row0 = sid * TROWS
        for k in range(TROWS // ZROWS):
            pltpu.sync_copy(rows0_v.at[pl.ds(0, ZROWS)],
                            agg_sh.at[pl.ds(tile_row0 + k * ZROWS, ZROWS)])
        wait_idx(0)
        bias_cols(0)
        issue_gathers(0, rows0_v)
        wait_idx(1)
        bias_cols(1)
        issue_gathers(1, rows1_v)
        plsc.subcore_barrier()

        _dnums = lax.GatherDimensionNumbers(
            offset_dims=(), collapsed_slice_dims=(0,), start_index_map=(0,))

        def scale(c, buf):
            slot = lax.rem(c, NRING)
            nj = DH // 16

            def grp(g, carry2):
                a16 = a_v[slot, pl.ds(g * 16, 16)]
                for l0 in range(0, 16, 4):
                    es = [g * 16 + l0 + i for i in range(4)]
                    bcs = [
                        lax.gather(
                            a16, jnp.full((16, 1), l0 + i, jnp.int32),
                            _dnums, (1,),
                            mode=lax.GatherScatterMode.PROMISE_IN_BOUNDS)
                        for i in range(4)
                    ]
                    vals = [[buf[es[i], pl.ds(j * 16, 16)] for j in range(nj)]
                            for i in range(4)]
                    for i in range(4):
                        for j in range(nj):
                            buf[es[i], pl.ds(j * 16, 16)] = vals[i][j] * bcs[i]
                return carry2

            lax.fori_loop(0, CHUNK // 16, grp, 0)

        def body5(c5, carry):
            for b in range(NBUFS):
                c = c5 * NBUFS + b
                buf = rows_bufs[b]
                gbuf = rows_bufs[(b + 2) % NBUFS]
                wait_gathers(c, buf)

                @pl.when(c + 2 < NCHUNK)
                def _():
                    @pl.when(c >= 3)
                    def _():
                        wait_scatters(c - 3, gbuf)

                    wait_idx(c + 2)
                    bias_cols(c + 2)
                    issue_gathers(c + 2, gbuf)

                @pl.when(c + 3 < NCHUNK)
                def _():
                    issue_idx(c + 3)

                scale(c, buf)
                issue_scatters(c, buf)
            return carry

        lax.fori_loop(0, NCHUNK // NBUFS, body5, 0)
        for c in range(NCHUNK - 5, NCHUNK):
            wait_scatters(c, rows_bufs[c % NBUFS])

        plsc.subcore_barrier()
        for k in range(TROWS // ZROWS):
            r0 = tile_row0 + k * ZROWS
            pltpu.sync_copy(agg_sh.at[pl.ds(r0, ZROWS)],
                            rows0_v.at[pl.ds(0, ZROWS)])
            pltpu.sync_copy(rows0_v.at[pl.ds(0, ZROWS)],
                            out_hbm.at[cid, pl.ds(r0, ZROWS)])

    return sc_agg


_sc_agg = _build_sc_agg()

_MB = 2000


def _tc_body(agg_ref, w_ref, b_ref, g_ref, bt_ref, o_ref):
    acc = jnp.concatenate([agg_ref[0], agg_ref[1]], axis=-1)
    h = jnp.dot(acc, w_ref[...], preferred_element_type=jnp.float32)
    h = jnp.maximum(h + b_ref[...], 0.0)
    o_ref[...] = (g_ref[...] * h) / jnp.sqrt(jnp.float32(1.0 + 1e-3)) + bt_ref[...]


def _tc_finish(partials, W, b, gamma, beta):
    return pl.pallas_call(
        _tc_body,
        grid=(N_NODES // _MB,),
        in_specs=[
            pl.BlockSpec((2, _MB, DH), lambda i: (0, i, 0)),
            pl.BlockSpec((D, D), lambda i: (0, 0)),
            pl.BlockSpec((1, D), lambda i: (0, 0)),
            pl.BlockSpec((1, D), lambda i: (0, 0)),
            pl.BlockSpec((1, D), lambda i: (0, 0)),
        ],
        out_specs=pl.BlockSpec((_MB, D), lambda i: (i, 0)),
        out_shape=jax.ShapeDtypeStruct((N_NODES, D), jnp.float32),
    )(partials, W, b.reshape(1, D), gamma.reshape(1, D), beta.reshape(1, D))


def kernel(x, edge_index, a_values, W, b, gamma, beta):
    row = edge_index[0].astype(jnp.int32)
    col = edge_index[1].astype(jnp.int32)
    n_edges = row.shape[0]
    pad = E_PAD - n_edges
    ipad = jnp.zeros((pad,), jnp.int32)
    rowp = jnp.concatenate([row, ipad])
    colp = jnp.concatenate([col, ipad])
    ap = jnp.concatenate([a_values, jnp.zeros((pad,), jnp.float32)])
    xh = jnp.concatenate([x[:, :DH], x[:, DH:]], axis=0)
    partials = _sc_agg(xh, colp, rowp, ap)
    return _tc_finish(partials, W, b, gamma, beta)
